# Initial kernel scaffold; baseline (speedup 1.0000x reference)
import jax
import jax.numpy as jnp
from jax.experimental import pallas as pl


def kernel(hidden_states, gate_w, w1, w2, w3):
    raise NotImplementedError("write your pallas kernel here")


# speedup vs baseline: 1.2638x; 1.2638x over previous
import functools

import jax
import jax.numpy as jnp
from jax import lax
from jax.experimental import pallas as pl
from jax.experimental.pallas import tpu as pltpu

HIDDEN = 2048
FFN = 4096
NUM_EXPERTS = 8
TOP_K = 2
SEQ = 2048

M_BLK = 256
NV = SEQ * TOP_K // M_BLK + (NUM_EXPERTS - 1)
M_PAD = NV * M_BLK
N_BLK = 1024
NN = FFN // N_BLK


def _router_body(hs_ref, gate_ref, logits_ref, ids_ref, wts_ref):
    logits = jnp.dot(hs_ref[...], gate_ref[...],
                     preferred_element_type=jnp.float32)
    logits_ref[...] = logits
    col = lax.broadcasted_iota(jnp.int32, logits.shape, 1)
    m1 = jnp.max(logits, axis=1, keepdims=True)
    a1 = jnp.min(jnp.where(logits == m1, col, NUM_EXPERTS), axis=1,
                 keepdims=True)
    masked = jnp.where(col == a1, -jnp.inf, logits)
    m2 = jnp.max(masked, axis=1, keepdims=True)
    a2 = jnp.min(jnp.where(masked == m2, col, NUM_EXPERTS), axis=1,
                 keepdims=True)
    z = jnp.exp(m2 - m1)
    w2_ = z / (1.0 + z)
    w1_ = 1.0 / (1.0 + z)
    ids_ref[...] = jnp.concatenate([a1, a2], axis=1)
    wts_ref[...] = jnp.concatenate([w1_, w2_], axis=1)


def _router(hs2d, gate_w):
    return pl.pallas_call(
        _router_body,
        out_shape=(
            jax.ShapeDtypeStruct((SEQ, NUM_EXPERTS), jnp.float32),
            jax.ShapeDtypeStruct((SEQ, TOP_K), jnp.int32),
            jax.ShapeDtypeStruct((SEQ, TOP_K), jnp.float32),
        ),
    )(hs2d, gate_w)


def _gmm_body(meta_ref, x_ref, w1_ref, w3_ref, w2_ref, wcol_ref, out_ref):
    n = pl.program_id(1)
    x = x_ref[...]
    h1 = jnp.dot(x, w1_ref[0], preferred_element_type=jnp.float32)
    h3 = jnp.dot(x, w3_ref[0], preferred_element_type=jnp.float32)
    h = (h1 * jax.nn.sigmoid(h1)) * h3
    y = jnp.dot(h, w2_ref[0], preferred_element_type=jnp.float32)

    @pl.when(n == 0)
    def _():
        out_ref[...] = y

    @pl.when(n > 0)
    def _():
        out_ref[...] += y

    @pl.when(n == NN - 1)
    def _():
        out_ref[...] *= wcol_ref[...]


def _gmm(x_pad, w1, w3, w2, w_col, tile_expert):
    grid_spec = pltpu.PrefetchScalarGridSpec(
        num_scalar_prefetch=1,
        grid=(NV, NN),
        in_specs=[
            pl.BlockSpec((M_BLK, HIDDEN), lambda v, n, m: (v, 0)),
            pl.BlockSpec((1, HIDDEN, N_BLK), lambda v, n, m: (m[v], 0, n)),
            pl.BlockSpec((1, HIDDEN, N_BLK), lambda v, n, m: (m[v], 0, n)),
            pl.BlockSpec((1, N_BLK, HIDDEN), lambda v, n, m: (m[v], n, 0)),
            pl.BlockSpec((M_BLK, 1), lambda v, n, m: (v, 0)),
        ],
        out_specs=pl.BlockSpec((M_BLK, HIDDEN), lambda v, n, m: (v, 0)),
    )
    return pl.pallas_call(
        _gmm_body,
        grid_spec=grid_spec,
        out_shape=jax.ShapeDtypeStruct((M_PAD, HIDDEN), jnp.float32),
        compiler_params=pltpu.CompilerParams(
            dimension_semantics=("arbitrary", "arbitrary"),
        ),
    )(tile_expert, x_pad, w1, w3, w2, w_col)


def kernel(hidden_states, gate_w, w1, w2, w3):
    hs2d = hidden_states.reshape(-1, HIDDEN)

    router_logits, ids, wts = _router(hs2d, gate_w)

    e_flat = ids.reshape(-1)
    onehot = (e_flat[:, None] == jnp.arange(NUM_EXPERTS)[None, :]).astype(
        jnp.int32)
    csum = jnp.cumsum(onehot, axis=0)
    rank = jnp.take_along_axis(csum, e_flat[:, None], axis=1)[:, 0] - 1
    counts = csum[-1]
    tiles_e = (counts + M_BLK - 1) // M_BLK
    tile_start = jnp.concatenate(
        [jnp.zeros((1,), jnp.int32), jnp.cumsum(tiles_e)[:-1]])
    pos = tile_start[e_flat] * M_BLK + rank
    src_row = jnp.zeros((M_PAD,), jnp.int32).at[pos].set(
        jnp.arange(SEQ * TOP_K, dtype=jnp.int32) // TOP_K)
    w_col = jnp.zeros((M_PAD,), jnp.float32).at[pos].set(
        wts.reshape(-1)).reshape(M_PAD, 1)
    tile_expert = jnp.minimum(
        jnp.searchsorted(jnp.cumsum(tiles_e), jnp.arange(NV), side="right"),
        NUM_EXPERTS - 1).astype(jnp.int32)

    x_pad = jnp.take(hs2d, src_row, axis=0)

    yw = _gmm(x_pad, w1, w3, w2, w_col, tile_expert)

    p = pos.reshape(SEQ, TOP_K)
    final = jnp.take(yw, p[:, 0], axis=0) + jnp.take(yw, p[:, 1], axis=0)
    return final, router_logits


# speedup vs baseline: 1.3616x; 1.0775x over previous
import functools

import jax
import jax.numpy as jnp
from jax import lax
from jax.experimental import pallas as pl
from jax.experimental.pallas import tpu as pltpu

HIDDEN = 2048
FFN = 4096
NUM_EXPERTS = 8
TOP_K = 2
SEQ = 2048

M_BLK = 256
NV = SEQ * TOP_K // M_BLK + (NUM_EXPERTS - 1)
M_PAD = NV * M_BLK
N_BLK = 1024
NN = FFN // N_BLK


def _router_body(hs_ref, gate_ref, logits_ref, ids_ref, wts_ref):
    logits = jnp.dot(hs_ref[...], gate_ref[...],
                     preferred_element_type=jnp.float32)
    logits_ref[...] = logits
    col = lax.broadcasted_iota(jnp.int32, logits.shape, 1)
    m1 = jnp.max(logits, axis=1, keepdims=True)
    a1 = jnp.min(jnp.where(logits == m1, col, NUM_EXPERTS), axis=1,
                 keepdims=True)
    masked = jnp.where(col == a1, -jnp.inf, logits)
    m2 = jnp.max(masked, axis=1, keepdims=True)
    a2 = jnp.min(jnp.where(masked == m2, col, NUM_EXPERTS), axis=1,
                 keepdims=True)
    z = jnp.exp(m2 - m1)
    w2_ = z / (1.0 + z)
    w1_ = 1.0 / (1.0 + z)
    ids_ref[...] = jnp.concatenate([a1, a2], axis=1)
    wts_ref[...] = jnp.concatenate([w1_, w2_], axis=1)


def _router(hs2d, gate_w):
    return pl.pallas_call(
        _router_body,
        out_shape=(
            jax.ShapeDtypeStruct((SEQ, NUM_EXPERTS), jnp.float32),
            jax.ShapeDtypeStruct((SEQ, TOP_K), jnp.int32),
            jax.ShapeDtypeStruct((SEQ, TOP_K), jnp.float32),
        ),
    )(hs2d, gate_w)


def _k1_body(meta_ref, x_ref, w1_ref, w3_ref, h_ref):
    x = x_ref[...]
    h1 = jnp.dot(x, w1_ref[0], preferred_element_type=jnp.float32)
    h3 = jnp.dot(x, w3_ref[0], preferred_element_type=jnp.float32)
    h_ref[...] = (h1 * jax.nn.sigmoid(h1)) * h3


def _k1(x_pad, w1, w3, tile_expert):
    grid_spec = pltpu.PrefetchScalarGridSpec(
        num_scalar_prefetch=1,
        grid=(NN, NV),
        in_specs=[
            pl.BlockSpec((M_BLK, HIDDEN), lambda n, v, m: (v, 0)),
            pl.BlockSpec((1, HIDDEN, N_BLK), lambda n, v, m: (m[v], 0, n)),
            pl.BlockSpec((1, HIDDEN, N_BLK), lambda n, v, m: (m[v], 0, n)),
        ],
        out_specs=pl.BlockSpec((M_BLK, N_BLK), lambda n, v, m: (v, n)),
    )
    return pl.pallas_call(
        _k1_body,
        grid_spec=grid_spec,
        out_shape=jax.ShapeDtypeStruct((M_PAD, FFN), jnp.float32),
        compiler_params=pltpu.CompilerParams(
            dimension_semantics=("arbitrary", "arbitrary"),
        ),
    )(tile_expert, x_pad, w1, w3)


N2_BLK = 2048
NK2 = FFN // N2_BLK


def _k2_body(meta_ref, h_ref, w2_ref, wcol_ref, out_ref):
    k = pl.program_id(1)
    y = jnp.dot(h_ref[...], w2_ref[0], preferred_element_type=jnp.float32)

    @pl.when(k == 0)
    def _():
        out_ref[...] = y

    @pl.when(k == NK2 - 1)
    def _():
        prev = y if NK2 == 1 else out_ref[...] + y
        out_ref[...] = prev * wcol_ref[...]


def _k2(h, w2, w_col, tile_expert):
    grid_spec = pltpu.PrefetchScalarGridSpec(
        num_scalar_prefetch=1,
        grid=(NV, NK2),
        in_specs=[
            pl.BlockSpec((M_BLK, N2_BLK), lambda v, k, m: (v, k)),
            pl.BlockSpec((1, N2_BLK, HIDDEN), lambda v, k, m: (m[v], k, 0)),
            pl.BlockSpec((M_BLK, 1), lambda v, k, m: (v, 0)),
        ],
        out_specs=pl.BlockSpec((M_BLK, HIDDEN), lambda v, k, m: (v, 0)),
    )
    return pl.pallas_call(
        _k2_body,
        grid_spec=grid_spec,
        out_shape=jax.ShapeDtypeStruct((M_PAD, HIDDEN), jnp.float32),
        compiler_params=pltpu.CompilerParams(
            dimension_semantics=("arbitrary", "arbitrary"),
        ),
    )(tile_expert, h, w2, w_col)


def kernel(hidden_states, gate_w, w1, w2, w3):
    hs2d = hidden_states.reshape(-1, HIDDEN)

    router_logits, ids, wts = _router(hs2d, gate_w)

    e_flat = ids.reshape(-1)
    onehot = (e_flat[:, None] == jnp.arange(NUM_EXPERTS)[None, :]).astype(
        jnp.int32)
    csum = jnp.cumsum(onehot, axis=0)
    rank = jnp.take_along_axis(csum, e_flat[:, None], axis=1)[:, 0] - 1
    counts = csum[-1]
    tiles_e = (counts + M_BLK - 1) // M_BLK
    tile_start = jnp.concatenate(
        [jnp.zeros((1,), jnp.int32), jnp.cumsum(tiles_e)[:-1]])
    pos = tile_start[e_flat] * M_BLK + rank
    src_row = jnp.zeros((M_PAD,), jnp.int32).at[pos].set(
        jnp.arange(SEQ * TOP_K, dtype=jnp.int32) // TOP_K)
    w_col = jnp.zeros((M_PAD,), jnp.float32).at[pos].set(
        wts.reshape(-1)).reshape(M_PAD, 1)
    tile_expert = jnp.minimum(
        jnp.searchsorted(jnp.cumsum(tiles_e), jnp.arange(NV), side="right"),
        NUM_EXPERTS - 1).astype(jnp.int32)

    x_pad = jnp.take(hs2d, src_row, axis=0)

    h = _k1(x_pad, w1, w3, tile_expert)
    yw = _k2(h, w2, w_col, tile_expert)

    p = pos.reshape(SEQ, TOP_K)
    final = jnp.take(yw, p[:, 0], axis=0) + jnp.take(yw, p[:, 1], axis=0)
    return final, router_logits


# speedup vs baseline: 1.3688x; 1.0053x over previous
import functools

import jax
import jax.numpy as jnp
from jax import lax
from jax.experimental import pallas as pl
from jax.experimental.pallas import tpu as pltpu

HIDDEN = 2048
FFN = 4096
NUM_EXPERTS = 8
TOP_K = 2
SEQ = 2048

M_BLK = 256
NV = SEQ * TOP_K // M_BLK + (NUM_EXPERTS - 1)
M_PAD = NV * M_BLK
N_BLK = 1024
NN = FFN // N_BLK


def _router_body(hs_ref, gate_ref, logits_ref, ids_ref, wts_ref):
    logits = jnp.dot(hs_ref[...], gate_ref[...],
                     preferred_element_type=jnp.float32)
    logits_ref[...] = logits
    col = lax.broadcasted_iota(jnp.int32, logits.shape, 1)
    m1 = jnp.max(logits, axis=1, keepdims=True)
    a1 = jnp.min(jnp.where(logits == m1, col, NUM_EXPERTS), axis=1,
                 keepdims=True)
    masked = jnp.where(col == a1, -jnp.inf, logits)
    m2 = jnp.max(masked, axis=1, keepdims=True)
    a2 = jnp.min(jnp.where(masked == m2, col, NUM_EXPERTS), axis=1,
                 keepdims=True)
    z = jnp.exp(m2 - m1)
    w2_ = z / (1.0 + z)
    w1_ = 1.0 / (1.0 + z)
    ids_ref[...] = jnp.concatenate([a1, a2], axis=1)
    wts_ref[...] = jnp.concatenate([w1_, w2_], axis=1)


def _router(hs2d, gate_w):
    return pl.pallas_call(
        _router_body,
        out_shape=(
            jax.ShapeDtypeStruct((SEQ, NUM_EXPERTS), jnp.float32),
            jax.ShapeDtypeStruct((SEQ, TOP_K), jnp.int32),
            jax.ShapeDtypeStruct((SEQ, TOP_K), jnp.float32),
        ),
    )(hs2d, gate_w)


def _k1_body(meta_ref, x_ref, w1_ref, w3_ref, h_ref):
    x = x_ref[...].astype(jnp.bfloat16)
    h1 = jnp.dot(x, w1_ref[0].astype(jnp.bfloat16),
                 preferred_element_type=jnp.float32)
    h3 = jnp.dot(x, w3_ref[0].astype(jnp.bfloat16),
                 preferred_element_type=jnp.float32)
    h_ref[...] = (h1 * jax.nn.sigmoid(h1)) * h3


def _k1(x_pad, w1, w3, tile_expert):
    grid_spec = pltpu.PrefetchScalarGridSpec(
        num_scalar_prefetch=1,
        grid=(NN, NV),
        in_specs=[
            pl.BlockSpec((M_BLK, HIDDEN), lambda n, v, m: (v, 0)),
            pl.BlockSpec((1, HIDDEN, N_BLK), lambda n, v, m: (m[v], 0, n)),
            pl.BlockSpec((1, HIDDEN, N_BLK), lambda n, v, m: (m[v], 0, n)),
        ],
        out_specs=pl.BlockSpec((M_BLK, N_BLK), lambda n, v, m: (v, n)),
    )
    return pl.pallas_call(
        _k1_body,
        grid_spec=grid_spec,
        out_shape=jax.ShapeDtypeStruct((M_PAD, FFN), jnp.float32),
        compiler_params=pltpu.CompilerParams(
            dimension_semantics=("arbitrary", "arbitrary"),
        ),
    )(tile_expert, x_pad, w1, w3)


N2_BLK = 2048
NK2 = FFN // N2_BLK


def _k2_body(meta_ref, h_ref, w2_ref, wcol_ref, out_ref):
    k = pl.program_id(1)
    y = jnp.dot(h_ref[...].astype(jnp.bfloat16),
                w2_ref[0].astype(jnp.bfloat16),
                preferred_element_type=jnp.float32)

    @pl.when(k == 0)
    def _():
        out_ref[...] = y

    @pl.when(k == NK2 - 1)
    def _():
        prev = y if NK2 == 1 else out_ref[...] + y
        out_ref[...] = prev * wcol_ref[...]


def _k2(h, w2, w_col, tile_expert):
    grid_spec = pltpu.PrefetchScalarGridSpec(
        num_scalar_prefetch=1,
        grid=(NV, NK2),
        in_specs=[
            pl.BlockSpec((M_BLK, N2_BLK), lambda v, k, m: (v, k)),
            pl.BlockSpec((1, N2_BLK, HIDDEN), lambda v, k, m: (m[v], k, 0)),
            pl.BlockSpec((M_BLK, 1), lambda v, k, m: (v, 0)),
        ],
        out_specs=pl.BlockSpec((M_BLK, HIDDEN), lambda v, k, m: (v, 0)),
    )
    return pl.pallas_call(
        _k2_body,
        grid_spec=grid_spec,
        out_shape=jax.ShapeDtypeStruct((M_PAD, HIDDEN), jnp.float32),
        compiler_params=pltpu.CompilerParams(
            dimension_semantics=("arbitrary", "arbitrary"),
        ),
    )(tile_expert, h, w2, w_col)


def kernel(hidden_states, gate_w, w1, w2, w3):
    hs2d = hidden_states.reshape(-1, HIDDEN)

    router_logits, ids, wts = _router(hs2d, gate_w)

    e_flat = ids.reshape(-1)
    onehot = (e_flat[:, None] == jnp.arange(NUM_EXPERTS)[None, :]).astype(
        jnp.int32)
    csum = jnp.cumsum(onehot, axis=0)
    rank = jnp.take_along_axis(csum, e_flat[:, None], axis=1)[:, 0] - 1
    counts = csum[-1]
    tiles_e = (counts + M_BLK - 1) // M_BLK
    tile_start = jnp.concatenate(
        [jnp.zeros((1,), jnp.int32), jnp.cumsum(tiles_e)[:-1]])
    pos = tile_start[e_flat] * M_BLK + rank
    src_row = jnp.zeros((M_PAD,), jnp.int32).at[pos].set(
        jnp.arange(SEQ * TOP_K, dtype=jnp.int32) // TOP_K)
    w_col = jnp.zeros((M_PAD,), jnp.float32).at[pos].set(
        wts.reshape(-1)).reshape(M_PAD, 1)
    tile_expert = jnp.minimum(
        jnp.searchsorted(jnp.cumsum(tiles_e), jnp.arange(NV), side="right"),
        NUM_EXPERTS - 1).astype(jnp.int32)

    x_pad = jnp.take(hs2d, src_row, axis=0)

    h = _k1(x_pad, w1, w3, tile_expert)
    yw = _k2(h, w2, w_col, tile_expert)

    p = pos.reshape(SEQ, TOP_K)
    final = jnp.take(yw, p[:, 0], axis=0) + jnp.take(yw, p[:, 1], axis=0)
    return final, router_logits


# speedup vs baseline: 1.3900x; 1.0155x over previous
import functools

import jax
import jax.numpy as jnp
from jax import lax
from jax.experimental import pallas as pl
from jax.experimental.pallas import tpu as pltpu

HIDDEN = 2048
FFN = 4096
NUM_EXPERTS = 8
TOP_K = 2
SEQ = 2048

M_BLK = 256
NV = SEQ * TOP_K // M_BLK + (NUM_EXPERTS - 1)
M_PAD = NV * M_BLK
N_BLK = 1024
NN = FFN // N_BLK


def _router_body(hs_ref, gate_ref, logits_ref, ids_ref, wts_ref):
    logits = jnp.dot(hs_ref[...], gate_ref[...],
                     preferred_element_type=jnp.float32)
    logits_ref[...] = logits
    col = lax.broadcasted_iota(jnp.int32, logits.shape, 1)
    m1 = jnp.max(logits, axis=1, keepdims=True)
    a1 = jnp.min(jnp.where(logits == m1, col, NUM_EXPERTS), axis=1,
                 keepdims=True)
    masked = jnp.where(col == a1, -jnp.inf, logits)
    m2 = jnp.max(masked, axis=1, keepdims=True)
    a2 = jnp.min(jnp.where(masked == m2, col, NUM_EXPERTS), axis=1,
                 keepdims=True)
    z = jnp.exp(m2 - m1)
    w2_ = z / (1.0 + z)
    w1_ = 1.0 / (1.0 + z)
    ids_ref[...] = jnp.concatenate([a1, a2], axis=1)
    wts_ref[...] = jnp.concatenate([w1_, w2_], axis=1)


def _router(hs2d, gate_w):
    return pl.pallas_call(
        _router_body,
        out_shape=(
            jax.ShapeDtypeStruct((SEQ, NUM_EXPERTS), jnp.float32),
            jax.ShapeDtypeStruct((SEQ, TOP_K), jnp.int32),
            jax.ShapeDtypeStruct((SEQ, TOP_K), jnp.float32),
        ),
    )(hs2d, gate_w)


def _k1_body(meta_ref, x_ref, w1_ref, w3_ref, h_ref):
    x = x_ref[...]
    h1 = jnp.dot(x, w1_ref[0].astype(jnp.bfloat16),
                 preferred_element_type=jnp.float32)
    h3 = jnp.dot(x, w3_ref[0].astype(jnp.bfloat16),
                 preferred_element_type=jnp.float32)
    h_ref[...] = ((h1 * jax.nn.sigmoid(h1)) * h3).astype(jnp.bfloat16)


def _k1(x_pad, w1, w3, tile_expert):
    grid_spec = pltpu.PrefetchScalarGridSpec(
        num_scalar_prefetch=1,
        grid=(NN, NV),
        in_specs=[
            pl.BlockSpec((M_BLK, HIDDEN), lambda n, v, m: (v, 0)),
            pl.BlockSpec((1, HIDDEN, N_BLK), lambda n, v, m: (m[v], 0, n)),
            pl.BlockSpec((1, HIDDEN, N_BLK), lambda n, v, m: (m[v], 0, n)),
        ],
        out_specs=pl.BlockSpec((M_BLK, N_BLK), lambda n, v, m: (v, n)),
    )
    return pl.pallas_call(
        _k1_body,
        grid_spec=grid_spec,
        out_shape=jax.ShapeDtypeStruct((M_PAD, FFN), jnp.bfloat16),
        compiler_params=pltpu.CompilerParams(
            dimension_semantics=("arbitrary", "arbitrary"),
        ),
    )(tile_expert, x_pad, w1, w3)


N2_BLK = 2048
NK2 = FFN // N2_BLK


def _k2_body(meta_ref, h_ref, w2_ref, wcol_ref, y0_ref, y1_ref):
    k = pl.program_id(0)
    y = jnp.dot(h_ref[...], w2_ref[0].astype(jnp.bfloat16),
                preferred_element_type=jnp.float32) * wcol_ref[...]

    @pl.when(k == 0)
    def _():
        y0_ref[...] = y

    @pl.when(k == 1)
    def _():
        y1_ref[...] = y


def _k2(h, w2, w_col, tile_expert):
    grid_spec = pltpu.PrefetchScalarGridSpec(
        num_scalar_prefetch=1,
        grid=(NK2, NV),
        in_specs=[
            pl.BlockSpec((M_BLK, N2_BLK), lambda k, v, m: (v, k)),
            pl.BlockSpec((1, N2_BLK, HIDDEN), lambda k, v, m: (m[v], k, 0)),
            pl.BlockSpec((M_BLK, 1), lambda k, v, m: (v, 0)),
        ],
        out_specs=[
            pl.BlockSpec((M_BLK, HIDDEN), lambda k, v, m: (v, 0)),
            pl.BlockSpec((M_BLK, HIDDEN), lambda k, v, m: (v, 0)),
        ],
    )
    return pl.pallas_call(
        _k2_body,
        grid_spec=grid_spec,
        out_shape=[
            jax.ShapeDtypeStruct((M_PAD, HIDDEN), jnp.float32),
            jax.ShapeDtypeStruct((M_PAD, HIDDEN), jnp.float32),
        ],
        compiler_params=pltpu.CompilerParams(
            dimension_semantics=("arbitrary", "arbitrary"),
        ),
    )(tile_expert, h, w2, w_col)


def kernel(hidden_states, gate_w, w1, w2, w3):
    hs2d = hidden_states.reshape(-1, HIDDEN)

    router_logits, ids, wts = _router(hs2d, gate_w)

    e_flat = ids.reshape(-1)
    onehot = (e_flat[:, None] == jnp.arange(NUM_EXPERTS)[None, :]).astype(
        jnp.int32)
    csum = jnp.cumsum(onehot, axis=0)
    rank = jnp.take_along_axis(csum, e_flat[:, None], axis=1)[:, 0] - 1
    counts = csum[-1]
    tiles_e = (counts + M_BLK - 1) // M_BLK
    tile_start = jnp.concatenate(
        [jnp.zeros((1,), jnp.int32), jnp.cumsum(tiles_e)[:-1]])
    pos = tile_start[e_flat] * M_BLK + rank
    src_row = jnp.zeros((M_PAD,), jnp.int32).at[pos].set(
        jnp.arange(SEQ * TOP_K, dtype=jnp.int32) // TOP_K)
    w_col = jnp.zeros((M_PAD,), jnp.float32).at[pos].set(
        wts.reshape(-1)).reshape(M_PAD, 1)
    tile_expert = jnp.minimum(
        jnp.searchsorted(jnp.cumsum(tiles_e), jnp.arange(NV), side="right"),
        NUM_EXPERTS - 1).astype(jnp.int32)

    x_pad = jnp.take(hs2d.astype(jnp.bfloat16), src_row, axis=0)

    h = _k1(x_pad, w1, w3, tile_expert)
    y0, y1 = _k2(h, w2, w_col, tile_expert)

    p = pos.reshape(SEQ, TOP_K)
    final = ((jnp.take(y0, p[:, 0], axis=0) + jnp.take(y1, p[:, 0], axis=0))
             + (jnp.take(y0, p[:, 1], axis=0) + jnp.take(y1, p[:, 1], axis=0)))
    return final, router_logits


# speedup vs baseline: 1.4384x; 1.0348x over previous
import functools

import jax
import jax.numpy as jnp
from jax import lax
from jax.experimental import pallas as pl
from jax.experimental.pallas import tpu as pltpu

HIDDEN = 2048
FFN = 4096
NUM_EXPERTS = 8
TOP_K = 2
SEQ = 2048

M_BLK = 256
NV = SEQ * TOP_K // M_BLK + (NUM_EXPERTS - 1)
M_PAD = NV * M_BLK
N_BLK = 1024
NN = FFN // N_BLK


def _router_body(hs_ref, gate_ref, logits_ref, ids_ref, wts_ref):
    logits = jnp.dot(hs_ref[...], gate_ref[...],
                     preferred_element_type=jnp.float32)
    logits_ref[...] = logits
    col = lax.broadcasted_iota(jnp.int32, logits.shape, 1)
    m1 = jnp.max(logits, axis=1, keepdims=True)
    a1 = jnp.min(jnp.where(logits == m1, col, NUM_EXPERTS), axis=1,
                 keepdims=True)
    masked = jnp.where(col == a1, -jnp.inf, logits)
    m2 = jnp.max(masked, axis=1, keepdims=True)
    a2 = jnp.min(jnp.where(masked == m2, col, NUM_EXPERTS), axis=1,
                 keepdims=True)
    z = jnp.exp(m2 - m1)
    w2_ = z / (1.0 + z)
    w1_ = 1.0 / (1.0 + z)
    ids_ref[...] = jnp.concatenate([a1, a2], axis=1)
    wts_ref[...] = jnp.concatenate([w1_, w2_], axis=1)


def _router(hs2d, gate_w):
    return pl.pallas_call(
        _router_body,
        out_shape=(
            jax.ShapeDtypeStruct((SEQ, NUM_EXPERTS), jnp.float32),
            jax.ShapeDtypeStruct((SEQ, TOP_K), jnp.int32),
            jax.ShapeDtypeStruct((SEQ, TOP_K), jnp.float32),
        ),
    )(hs2d, gate_w)


def _k1_body(meta_ref, x_ref, w1_ref, w3_ref, h_ref):
    x = x_ref[...]
    h1 = jnp.dot(x, w1_ref[0].astype(jnp.bfloat16),
                 preferred_element_type=jnp.float32)
    h3 = jnp.dot(x, w3_ref[0].astype(jnp.bfloat16),
                 preferred_element_type=jnp.float32)
    h_ref[...] = ((h1 * jax.nn.sigmoid(h1)) * h3).astype(jnp.bfloat16)


def _k1(x_pad, w1, w3, tile_expert):
    grid_spec = pltpu.PrefetchScalarGridSpec(
        num_scalar_prefetch=1,
        grid=(NN, NV),
        in_specs=[
            pl.BlockSpec((M_BLK, HIDDEN), lambda n, v, m: (v, 0)),
            pl.BlockSpec((1, HIDDEN, N_BLK), lambda n, v, m: (m[v], 0, n)),
            pl.BlockSpec((1, HIDDEN, N_BLK), lambda n, v, m: (m[v], 0, n)),
        ],
        out_specs=pl.BlockSpec((M_BLK, N_BLK), lambda n, v, m: (v, n)),
    )
    return pl.pallas_call(
        _k1_body,
        grid_spec=grid_spec,
        out_shape=jax.ShapeDtypeStruct((M_PAD, FFN), jnp.bfloat16),
        compiler_params=pltpu.CompilerParams(
            dimension_semantics=("arbitrary", "arbitrary"),
        ),
    )(tile_expert, x_pad, w1, w3)


N2_BLK = 2048
NK2 = FFN // N2_BLK


def _k2_body(meta_ref, h_ref, w2_ref, wcol_ref, y_ref):
    y_ref[...] = jnp.dot(h_ref[...], w2_ref[0].astype(jnp.bfloat16),
                         preferred_element_type=jnp.float32) * wcol_ref[...]


def _k2(h, w2, w_col, tile_expert):
    grid_spec = pltpu.PrefetchScalarGridSpec(
        num_scalar_prefetch=1,
        grid=(NK2, NV),
        in_specs=[
            pl.BlockSpec((M_BLK, N2_BLK), lambda k, v, m: (v, k)),
            pl.BlockSpec((1, N2_BLK, HIDDEN), lambda k, v, m: (m[v], k, 0)),
            pl.BlockSpec((M_BLK, 1), lambda k, v, m: (v, 0)),
        ],
        out_specs=pl.BlockSpec((M_BLK, HIDDEN), lambda k, v, m: (k * NV + v, 0)),
    )
    return pl.pallas_call(
        _k2_body,
        grid_spec=grid_spec,
        out_shape=jax.ShapeDtypeStruct((NK2 * M_PAD, HIDDEN), jnp.float32),
        compiler_params=pltpu.CompilerParams(
            dimension_semantics=("arbitrary", "arbitrary"),
        ),
    )(tile_expert, h, w2, w_col)


def kernel(hidden_states, gate_w, w1, w2, w3):
    hs2d = hidden_states.reshape(-1, HIDDEN)

    router_logits, ids, wts = _router(hs2d, gate_w)

    e_flat = ids.reshape(-1)
    onehot = (e_flat[:, None] == jnp.arange(NUM_EXPERTS)[None, :]).astype(
        jnp.int32)
    csum = jnp.cumsum(onehot, axis=0)
    rank = jnp.take_along_axis(csum, e_flat[:, None], axis=1)[:, 0] - 1
    counts = csum[-1]
    tiles_e = (counts + M_BLK - 1) // M_BLK
    tile_start = jnp.concatenate(
        [jnp.zeros((1,), jnp.int32), jnp.cumsum(tiles_e)[:-1]])
    pos = tile_start[e_flat] * M_BLK + rank
    src_row = jnp.zeros((M_PAD,), jnp.int32).at[pos].set(
        jnp.arange(SEQ * TOP_K, dtype=jnp.int32) // TOP_K)
    w_col = jnp.zeros((M_PAD,), jnp.float32).at[pos].set(
        wts.reshape(-1)).reshape(M_PAD, 1)
    tile_expert = jnp.minimum(
        jnp.searchsorted(jnp.cumsum(tiles_e), jnp.arange(NV), side="right"),
        NUM_EXPERTS - 1).astype(jnp.int32)

    x_pad = jnp.take(hs2d.astype(jnp.bfloat16), src_row, axis=0)

    h = _k1(x_pad, w1, w3, tile_expert)
    yw = _k2(h, w2, w_col, tile_expert)

    p = pos.reshape(SEQ, TOP_K)
    final = ((jnp.take(yw, p[:, 0], axis=0)
              + jnp.take(yw, p[:, 0] + M_PAD, axis=0))
             + (jnp.take(yw, p[:, 1], axis=0)
                + jnp.take(yw, p[:, 1] + M_PAD, axis=0)))
    return final, router_logits
